# baseline (speedup 1.0000x reference)
import functools
import jax
import jax.numpy as jnp
from jax import lax
from jax.experimental import pallas as pl
from jax.experimental.pallas import tpu as pltpu
from jax.experimental.pallas import tpu_sc as plsc

TOPK = 8
L = 16
RCHUNK = 8
NACC = 8


def _bfly(v, iota, op):
    for s in (8, 4, 2, 1):
        perm = jnp.bitwise_xor(iota, s)
        v = op(v, v.at[perm].get(mode="promise_in_bounds"))
    return v


def _make_sc_rowmean(B, C, S):
    NCH = C // RCHUNK
    NV = S // L
    inv_s = 1.0 / S

    mesh = plsc.VectorSubcoreMesh(core_axis_name="c", subcore_axis_name="s")

    @functools.partial(
        pl.kernel,
        mesh=mesh,
        out_type=jax.ShapeDtypeStruct((B, C), jnp.float32),
        scratch_types=[
            pltpu.VMEM((2, RCHUNK, S), jnp.float32),
            pltpu.VMEM((C,), jnp.float32),
            pltpu.SemaphoreType.DMA((2,)),
        ],
    )
    def sc_rowmean(x_hbm, out_hbm, buf, sums_v, sems):
        w = lax.axis_index("s") * 2 + lax.axis_index("c")
        iota = lax.broadcasted_iota(jnp.int32, (L,), 0)

        def cp(g, slot):
            return pltpu.make_async_copy(
                x_hbm.at[w, pl.ds(g * RCHUNK, RCHUNK), :],
                buf.at[slot], sems.at[slot])

        cp(0, 0).start()
        cp(1, 1).start()

        def chunk_body(g, rv):
            slot = lax.rem(g, 2)
            cp(g, slot).wait()
            for r in range(RCHUNK):
                accs = [buf[slot, r, pl.ds(u * L, L)] for u in range(NACC)]
                for j in range(NACC, NV):
                    accs[j % NACC] = (accs[j % NACC]
                                      + buf[slot, r, pl.ds(j * L, L)])
                a = ((accs[0] + accs[1]) + (accs[2] + accs[3])) + (
                    (accs[4] + accs[5]) + (accs[6] + accs[7]))
                acc = _bfly(a, iota, jnp.add)
                rv = jnp.where(iota == slot * RCHUNK + r, acc * inv_s, rv)

            @pl.when(lax.rem(g, 2) == 1)
            def _():
                sums_v[pl.ds((g // 2) * L, L)] = rv

            @pl.when(g + 2 < NCH)
            def _():
                cp(g + 2, lax.rem(g + 2, 2)).start()
            return rv

        lax.fori_loop(0, NCH, chunk_body, jnp.zeros((L,), jnp.float32))
        pltpu.sync_copy(sums_v, out_hbm.at[w])

    return sc_rowmean


def _tc_tail_body(xm_ref, gw_ref, gb_ref, eb_ref, probs_out, idx_out):
    xm = xm_ref[...]
    nrows, nexp = xm.shape[0], gw_ref.shape[0]
    logits = jax.lax.dot_general(
        xm, gw_ref[...], (((1,), (1,)), ((), ())),
        preferred_element_type=jnp.float32)
    logits = logits + gb_ref[...]
    logits = jnp.clip(logits, -10.0, 10.0)
    lb = logits + eb_ref[...]
    m = jnp.max(lb, axis=1, keepdims=True)
    e = jnp.exp(lb - m)
    p = e / jnp.sum(e, axis=1, keepdims=True)
    p = jnp.clip(p, 1e-06, 1.0)
    iota = jax.lax.broadcasted_iota(jnp.int32, (nrows, nexp), 1)
    vals, idxs = [], []
    for _ in range(TOPK):
        mk = jnp.max(p, axis=1, keepdims=True)
        ik = jnp.min(jnp.where(p == mk, iota, nexp), axis=1, keepdims=True)
        vals.append(mk)
        idxs.append(ik)
        p = jnp.where(iota == ik, -jnp.inf, p)
    tv = jnp.concatenate(vals, axis=1)
    ti = jnp.concatenate(idxs, axis=1)
    tv = tv / (jnp.sum(tv, axis=1, keepdims=True) + 1e-08)
    probs_out[...] = tv
    idx_out[...] = ti


def kernel(x, gate_w, gate_b, expert_bias):
    B, C, H, W = x.shape
    E = gate_w.shape[0]
    S = H * W
    xr = x.reshape(B, C, S)

    xm = _make_sc_rowmean(B, C, S)(xr)

    gb = gate_b.reshape(1, E)
    eb = expert_bias.reshape(1, E)
    probs, idx = pl.pallas_call(
        _tc_tail_body,
        in_specs=[
            pl.BlockSpec((B, C), lambda: (0, 0)),
            pl.BlockSpec((E, C), lambda: (0, 0)),
            pl.BlockSpec((1, E), lambda: (0, 0)),
            pl.BlockSpec((1, E), lambda: (0, 0)),
        ],
        out_specs=[
            pl.BlockSpec((B, TOPK), lambda: (0, 0)),
            pl.BlockSpec((B, TOPK), lambda: (0, 0)),
        ],
        out_shape=[
            jax.ShapeDtypeStruct((B, TOPK), jnp.float32),
            jax.ShapeDtypeStruct((B, TOPK), jnp.int32),
        ],
    )(xm, gate_w, gb, eb)

    loss = jnp.zeros((), dtype=jnp.float32)
    return (probs, idx, loss)
